# baseline (speedup 1.0000x reference)
import functools

import jax
import jax.numpy as jnp
from jax import lax
from jax.experimental import pallas as pl
from jax.experimental.pallas import tpu as pltpu
from jax.experimental.pallas import tpu_sc as plsc

N = 50000
RS = 0.9999950000374996

L = 128
K = 8
CH = K * L
W = 16
NP = 50048
RPT = NP // 16
DUMP = N
BN = 2000


@functools.lru_cache(maxsize=None)
def _agg_call(P, CHUNKS):
    mesh = plsc.VectorSubcoreMesh(core_axis_name="c", subcore_axis_name="s")
    out_type = tuple(jax.ShapeDtypeStruct((NP, W), jnp.float32)
                     for _ in range(P))
    scratch = [
        pltpu.VMEM((K, L), jnp.int32),
        pltpu.VMEM((K, L), jnp.int32),
        pltpu.VMEM((CH, W), jnp.float32),
        pltpu.VMEM_SHARED((NP, W), jnp.float32),
        pltpu.SemaphoreType.DMA,
    ]

    def body(src_hbm, dst_hbm, *rest):
        g_refs = rest[:P]
        s_refs = rest[P:2 * P]
        src_v, dst_v, rows_v, acc, sem = rest[2 * P:]
        cid = lax.axis_index("c")
        sid = lax.axis_index("s")
        for p in range(P):
            @pl.when(cid == (p % 2))
            def _(p=p):
                g = g_refs[p]
                pltpu.sync_copy(g.at[pl.ds(sid * RPT, RPT)],
                                acc.at[pl.ds(sid * RPT, RPT)])
                plsc.subcore_barrier()

                def chunk(j, carry):
                    r = sid * CHUNKS + j
                    pltpu.sync_copy(src_hbm.at[r], src_v)
                    pltpu.sync_copy(dst_hbm.at[r], dst_v)
                    handles = [
                        pltpu.async_copy(g.at[src_v.at[k]],
                                         rows_v.at[pl.ds(k * L, L)], sem)
                        for k in range(K)
                    ]
                    for h in handles:
                        h.wait()
                    for k in range(K):
                        pltpu.sync_copy(rows_v.at[pl.ds(k * L, L)],
                                        acc.at[dst_v.at[k]], add=True)
                    return carry

                lax.fori_loop(0, CHUNKS, chunk, 0)
                plsc.subcore_barrier()
                pltpu.sync_copy(acc.at[pl.ds(sid * RPT, RPT)],
                                s_refs[p].at[pl.ds(sid * RPT, RPT)])
                plsc.subcore_barrier()

    return pl.kernel(body, out_type=out_type, mesh=mesh,
                     scratch_types=scratch,
                     compiler_params=pltpu.CompilerParams(
                         use_tc_tiling_on_sc=False))


def _pad_edges(ei, tiles):
    e = ei.shape[1]
    per = -(-e // (tiles * CH)) * CH
    chunks = per // CH
    pad = tiles * per - e
    src = jnp.concatenate([ei[0], jnp.zeros((pad,), jnp.int32)])
    dst = jnp.concatenate([ei[1], jnp.full((pad,), DUMP, jnp.int32)])
    return (src.reshape(tiles * chunks, K, L),
            dst.reshape(tiles * chunks, K, L), chunks)


def _mm_scale_kernel(x_ref, w_ref, dinv_ref, o_ref):
    o_ref[...] = dinv_ref[...] * jnp.dot(
        x_ref[...], w_ref[...], preferred_element_type=jnp.float32)


def _mm_scale(x, wmat, dinv2):
    n, fin = x.shape
    fout = wmat.shape[1]
    return pl.pallas_call(
        _mm_scale_kernel,
        grid=(n // BN,),
        in_specs=[pl.BlockSpec((BN, fin), lambda i: (i, 0)),
                  pl.BlockSpec((fin, fout), lambda i: (0, 0)),
                  pl.BlockSpec((BN, 1), lambda i: (i, 0))],
        out_specs=pl.BlockSpec((BN, fout), lambda i: (i, 0)),
        out_shape=jax.ShapeDtypeStruct((n, fout), jnp.float32),
    )(x, wmat, dinv2)


def _post_mm_kernel(s_ref, dinv_ref, b_ref, c1_ref, c2_ref, w_ref, o_ref):
    h = jnp.maximum((dinv_ref[...] * s_ref[...] + b_ref[...]) * c1_ref[...]
                    + c2_ref[...], 0.0)
    o_ref[...] = dinv_ref[...] * jnp.dot(
        h, w_ref[...], preferred_element_type=jnp.float32)


def _post_mm(s, dinv2, b, c1, c2, wmat):
    n, fin = s.shape
    fout = wmat.shape[1]
    return pl.pallas_call(
        _post_mm_kernel,
        grid=(n // BN,),
        in_specs=[pl.BlockSpec((BN, fin), lambda i: (i, 0)),
                  pl.BlockSpec((BN, 1), lambda i: (i, 0)),
                  pl.BlockSpec((1, fin), lambda i: (0, 0)),
                  pl.BlockSpec((1, fin), lambda i: (0, 0)),
                  pl.BlockSpec((1, fin), lambda i: (0, 0)),
                  pl.BlockSpec((fin, fout), lambda i: (0, 0))],
        out_specs=pl.BlockSpec((BN, fout), lambda i: (i, 0)),
        out_shape=jax.ShapeDtypeStruct((n, fout), jnp.float32),
    )(s, dinv2, b, c1, c2, wmat)


def _pool_kernel(s_ref, dinv_ref, b_ref, c1_ref, c2_ref, v_ref, o_ref):
    h = jnp.maximum((dinv_ref[...] * s_ref[...] + b_ref[...]) * c1_ref[...]
                    + c2_ref[...], 0.0)
    part = jnp.sum(v_ref[...] * h, axis=0, keepdims=True)

    @pl.when(pl.program_id(0) == 0)
    def _():
        o_ref[...] = jnp.zeros_like(o_ref)

    o_ref[...] += part


def _pool(s, dinv2, b, c1, c2, v2):
    n, fin = s.shape
    return pl.pallas_call(
        _pool_kernel,
        grid=(n // BN,),
        in_specs=[pl.BlockSpec((BN, fin), lambda i: (i, 0)),
                  pl.BlockSpec((BN, 1), lambda i: (i, 0)),
                  pl.BlockSpec((1, fin), lambda i: (0, 0)),
                  pl.BlockSpec((1, fin), lambda i: (0, 0)),
                  pl.BlockSpec((1, fin), lambda i: (0, 0)),
                  pl.BlockSpec((BN, 1), lambda i: (i, 0))],
        out_specs=pl.BlockSpec((1, fin), lambda i: (0, 0)),
        out_shape=jax.ShapeDtypeStruct((1, fin), jnp.float32),
    )(s, dinv2, b, c1, c2, v2)


def _head_kernel(u0, u1, u2, w30, w31, w32, d30, d31, d32, e30, e31, e32,
                 wf_ref, bf_ref, wc_ref, bc_ref, wr_ref, br_ref,
                 logits_ref, reg_ref):
    embs = []
    for u, w3, d3, e3 in ((u0, w30, d30, e30), (u1, w31, d31, e31),
                          (u2, w32, d32, e32)):
        embs.append(jnp.dot(u[...], w3[...],
                            preferred_element_type=jnp.float32)
                    * d3[...] + e3[...])
    fused = jnp.concatenate(embs, axis=1)
    h = jnp.maximum(fused @ wf_ref[...] + bf_ref[...], 0.0)
    logits_ref[...] = h @ wc_ref[...] + bc_ref[...]
    reg_ref[...] = jax.nn.sigmoid(h @ wr_ref[...] + br_ref[...])


def _head(us, l3s, f):
    args = list(us)
    args += [p["W"] for p in l3s]
    args += [(RS * p["gamma"])[None, :] for p in l3s]
    args += [((RS * p["gamma"]) * p["b"] + p["beta"])[None, :] for p in l3s]
    args += [f["Wf"], f["bf"][None, :], f["Wc"], f["bc"][None, :],
             f["Wr"], f["br"][None, :]]
    return pl.pallas_call(
        _head_kernel,
        out_shape=(jax.ShapeDtypeStruct((1, 10), jnp.float32),
                   jax.ShapeDtypeStruct((1, 1), jnp.float32)),
    )(*args)


def _branch(x, ei, layers):
    src3, dst3, chunks = _pad_edges(ei, 16)

    deg = jnp.zeros((N,), jnp.float32).at[ei[1]].add(1.0) + 1.0
    dinv = lax.rsqrt(deg)
    w = jnp.zeros((N,), jnp.float32).at[ei[0]].add(dinv[ei[1]])
    v = (w + dinv) * dinv * (1.0 / N)

    dinv2 = dinv[:, None]
    v2 = v[:, None]

    g = _mm_scale(x, layers[0]["W"], dinv2)
    for i in range(2):
        p = layers[i]
        slices = g.shape[1] // W
        gp = jnp.pad(g, ((0, NP - N), (0, 0)))
        g_sl = [gp[:, W * q:W * q + W] for q in range(slices)]
        s_sl = _agg_call(slices, chunks)(src3, dst3, *g_sl)
        s = jnp.concatenate([t[:N] for t in s_sl], axis=1)
        c1 = (RS * p["gamma"])[None, :]
        if i == 0:
            g = _post_mm(s, dinv2, p["b"][None, :], c1, p["beta"][None, :],
                         layers[1]["W"])
        else:
            u = _pool(s, dinv2, p["b"][None, :], c1, p["beta"][None, :], v2)
    return u


def kernel(x, edge_index_s0, edge_index_s1, edge_index_s2, params):
    eis = [edge_index_s0, edge_index_s1, edge_index_s2]
    us = [_branch(x, eis[i], params["scales"][i]) for i in range(3)]
    l3s = [params["scales"][i][2] for i in range(3)]
    logits, reg = _head(us, l3s, params["fusion"])
    return (logits, reg)
